# baseline (speedup 1.0000x reference)
import jax
import jax.numpy as jnp
from jax.experimental import pallas as pl
from jax.experimental.pallas import tpu as pltpu


def _fused_kernel(x_ref, w_ref, b_ref, o_ref):
    x = x_ref[...].astype(jnp.bfloat16)
    y = jnp.dot(x, w_ref[...], preferred_element_type=jnp.float32)
    o_ref[...] = y + b_ref[...]


def kernel(encoded, weight, bias, *, tile_b=1024):
    B, C, H = encoded.shape
    Cw, Hw, F = weight.shape
    assert (C, H) == (Cw, Hw) and bias.shape == (C, F)

    x_flat = encoded.reshape(B, C * H)

    w_bd = jax.scipy.linalg.block_diag(*[weight[c] for c in range(C)])
    w_perm = (w_bd.reshape(C * H, C, F).transpose(0, 2, 1)
              .reshape(C * H, F * C).astype(jnp.bfloat16))
    b_perm = bias.T.reshape(1, F * C)

    tb = min(tile_b, B)
    pad = (-B) % tb
    if pad:
        x_flat = jnp.pad(x_flat, ((0, pad), (0, 0)))
    Bp = x_flat.shape[0]

    out_flat = pl.pallas_call(
        _fused_kernel,
        out_shape=jax.ShapeDtypeStruct((Bp, F * C), encoded.dtype),
        grid=(Bp // tb,),
        in_specs=[
            pl.BlockSpec((tb, C * H), lambda i: (i, 0)),
            pl.BlockSpec((C * H, F * C), lambda i: (0, 0)),
            pl.BlockSpec((1, F * C), lambda i: (0, 0)),
        ],
        out_specs=pl.BlockSpec((tb, F * C), lambda i: (i, 0)),
        compiler_params=pltpu.CompilerParams(
            dimension_semantics=("parallel",)),
    )(x_flat, w_perm, b_perm)

    return out_flat[:B].reshape(B, F, C)


# speedup vs baseline: 1.3094x; 1.3094x over previous
import jax
import jax.numpy as jnp
from jax.experimental import pallas as pl
from jax.experimental.pallas import tpu as pltpu


def _per_channel_kernel(x_ref, w_ref, b_ref, o_ref):
    C = w_ref.shape[0]
    F = w_ref.shape[2]
    for c in range(C):
        xc = x_ref[:, c, :].astype(jnp.bfloat16)
        wc = w_ref[c].astype(jnp.bfloat16)
        y = jnp.dot(xc, wc, preferred_element_type=jnp.float32)
        o_ref[:, c * F:(c + 1) * F] = y + b_ref[c, :]


def kernel(encoded, weight, bias, *, tile_b=1024):
    B, C, H = encoded.shape
    Cw, Hw, F = weight.shape
    assert (C, H) == (Cw, Hw) and bias.shape == (C, F)

    tb = min(tile_b, B)
    pad = (-B) % tb
    if pad:
        encoded = jnp.pad(encoded, ((0, pad), (0, 0), (0, 0)))
    Bp = encoded.shape[0]

    out_flat = pl.pallas_call(
        _per_channel_kernel,
        out_shape=jax.ShapeDtypeStruct((Bp, C * F), encoded.dtype),
        grid=(Bp // tb,),
        in_specs=[
            pl.BlockSpec((tb, C, H), lambda i: (i, 0, 0)),
            pl.BlockSpec((C, H, F), lambda i: (0, 0, 0)),
            pl.BlockSpec((C, F), lambda i: (0, 0)),
        ],
        out_specs=pl.BlockSpec((tb, C * F), lambda i: (i, 0)),
        compiler_params=pltpu.CompilerParams(
            dimension_semantics=("parallel",)),
    )(encoded, weight, bias)

    out_flat = out_flat[:B]
    return jnp.transpose(out_flat.reshape(B, C, F), (0, 2, 1))
